# Initial kernel scaffold; baseline (speedup 1.0000x reference)
import jax
import jax.numpy as jnp
from jax.experimental import pallas as pl


def kernel(count, val_emb, bit_emb):
    raise NotImplementedError("write your pallas kernel here")


# speedup vs baseline: 7.6963x; 7.6963x over previous
import jax
import jax.numpy as jnp
from jax.experimental import pallas as pl
from jax.experimental.pallas import tpu as pltpu

COUNT_DIM = 100
N_EMBD = 64
BATCH = 16384
ROWS = BATCH * COUNT_DIM
RBLK = 6400
KPAD = 128


def _body(cnt_ref, val_ref, bit_ref, out_ref, bit_tile_ref):
    lane = jax.lax.broadcasted_iota(jnp.int32, (RBLK, KPAD), 1)

    @pl.when(pl.program_id(0) == 0)
    def _init():
        d = jax.lax.broadcasted_iota(jnp.int32, (RBLK, KPAD), 0) % COUNT_DIM
        oh_d = jnp.where(d == lane, 1, 0).astype(jnp.bfloat16)
        bit_tile_ref[...] = jax.lax.dot_general(
            oh_d, bit_ref[...],
            (((1,), (0,)), ((), ())),
            preferred_element_type=jnp.float32)

    cnt = jnp.broadcast_to(cnt_ref[...], (RBLK, KPAD))
    oh = jnp.where(cnt == lane, 1, 0).astype(jnp.bfloat16)
    rows = jax.lax.dot_general(
        oh, val_ref[...],
        (((1,), (0,)), ((), ())),
        preferred_element_type=jnp.float32)
    out_ref[...] = rows + bit_tile_ref[...]


def kernel(count, val_emb, bit_emb):
    cnt_flat = count.astype(jnp.int32).reshape(ROWS, 1)
    val_pad = jnp.pad(val_emb, ((0, KPAD - 100), (0, 0))).astype(jnp.bfloat16)
    bit_pad = jnp.pad(bit_emb, ((0, KPAD - COUNT_DIM), (0, 0))).astype(jnp.bfloat16)
    out2 = pl.pallas_call(
        _body,
        grid=(ROWS // RBLK,),
        in_specs=[
            pl.BlockSpec((RBLK, 1), lambda i: (i, 0)),
            pl.BlockSpec((KPAD, N_EMBD), lambda i: (0, 0)),
            pl.BlockSpec((KPAD, N_EMBD), lambda i: (0, 0)),
        ],
        out_specs=pl.BlockSpec((RBLK, N_EMBD), lambda i: (i, 0)),
        out_shape=jax.ShapeDtypeStruct((ROWS, N_EMBD), jnp.float32),
        scratch_shapes=[pltpu.VMEM((RBLK, N_EMBD), jnp.float32)],
    )(cnt_flat, val_pad, bit_pad)
    return out2.reshape(BATCH, COUNT_DIM, N_EMBD)


# speedup vs baseline: 8.5076x; 1.1054x over previous
import jax
import jax.numpy as jnp
from jax import lax
from jax.experimental import pallas as pl
from jax.experimental.pallas import tpu as pltpu
from jax.experimental.pallas import tpu_sc as plsc

COUNT_DIM = 100
N_EMBD = 64
BATCH = 16384
NVALS = 100

L = 16
NC = 2
NS = 16
NW = NC * NS
CB = 512
NCHUNK = BATCH // CB
ITEMS = COUNT_DIM * NCHUNK
IPW = ITEMS // NW

_DNUMS = lax.GatherDimensionNumbers(
    offset_dims=(), collapsed_slice_dims=(0,), start_index_map=(0,))


def _lane_bcast(vec, e):
    idx = jnp.full((L, 1), e, jnp.int32)
    return lax.gather(vec, idx, _DNUMS, (1,),
                      mode=lax.GatherScatterMode.PROMISE_IN_BOUNDS)


def _sc_body(cntT_hbm, val_hbm, bit_hbm, out_hbm, val_v, bit_v, cnt_v, ob):
    wid = lax.axis_index("s") * NC + lax.axis_index("c")

    pltpu.sync_copy(val_hbm, val_v)
    pltpu.sync_copy(bit_hbm, bit_v)

    col0 = lax.iota(jnp.int32, L)

    def item_body(t, carry):
        item = wid * IPW + t
        d = item // NCHUNK
        ch = item - d * NCHUNK
        b0 = ch * CB

        pltpu.sync_copy(cntT_hbm.at[d, pl.ds(b0, CB)], cnt_v)

        bits = [bit_v[pl.ds(d * N_EMBD + L * j, L)] for j in range(4)]

        def group_body(g, carry2):
            cvec = cnt_v[pl.ds(g * L, L)]
            for e in range(L):
                ridx = _lane_bcast(cvec, e) * N_EMBD
                for j in range(4):
                    vals = plsc.load_gather(val_v, [ridx + (16 * j) + col0])
                    ob[g * L + e, pl.ds(16 * j, L)] = vals + bits[j]
            return carry2

        lax.fori_loop(0, CB // L, group_body, 0, unroll=False)

        pltpu.sync_copy(ob, out_hbm.at[pl.ds(b0, CB), d])
        return carry

    lax.fori_loop(0, IPW, item_body, 0, unroll=False)


def kernel(count, val_emb, bit_emb):
    cnt_t = count.astype(jnp.int32).T
    val_flat = val_emb.reshape(-1)
    bit_flat = bit_emb.reshape(-1)

    mesh = plsc.VectorSubcoreMesh(core_axis_name="c", subcore_axis_name="s")
    f = pl.kernel(
        _sc_body,
        mesh=mesh,
        compiler_params=pltpu.CompilerParams(needs_layout_passes=False),
        out_type=jax.ShapeDtypeStruct((BATCH, COUNT_DIM, N_EMBD), jnp.float32),
        scratch_types=[
            pltpu.VMEM((NVALS * N_EMBD,), jnp.float32),
            pltpu.VMEM((COUNT_DIM * N_EMBD,), jnp.float32),
            pltpu.VMEM((CB,), jnp.int32),
            pltpu.VMEM((CB, N_EMBD), jnp.float32),
        ],
    )
    return f(cnt_t, val_flat, bit_flat)


# speedup vs baseline: 15.0601x; 1.7702x over previous
import jax
import jax.numpy as jnp
from jax import lax
from jax.experimental import pallas as pl
from jax.experimental.pallas import tpu as pltpu
from jax.experimental.pallas import tpu_sc as plsc

COUNT_DIM = 100
N_EMBD = 64
BATCH = 16384
NVALS = 100

L = 16
NC = 2
NS = 16
NW = NC * NS
CB = 256
NCHUNK = BATCH // CB
ITEMS = COUNT_DIM * NCHUNK
IPW = ITEMS // NW

_DNUMS = lax.GatherDimensionNumbers(
    offset_dims=(), collapsed_slice_dims=(0,), start_index_map=(0,))


def _lane_bcast(vec, e):
    idx = jnp.full((L, 1), e, jnp.int32)
    return lax.gather(vec, idx, _DNUMS, (1,),
                      mode=lax.GatherScatterMode.PROMISE_IN_BOUNDS)


def _sc_body(cntT_hbm, val_hbm, bit_hbm, out_hbm,
             val_v, bit_v, cnt_v, ob0, ob1, sem0, sem1):
    wid = lax.axis_index("s") * NC + lax.axis_index("c")

    pltpu.sync_copy(val_hbm, val_v)
    pltpu.sync_copy(bit_hbm, bit_v)

    col0 = lax.iota(jnp.int32, L)

    def do_item(t, ob, sem, first):
        item = wid * IPW + t
        d = item // NCHUNK
        ch = item - d * NCHUNK
        b0 = ch * CB

        pltpu.sync_copy(cntT_hbm.at[d, pl.ds(b0, CB)], cnt_v)

        bits = [bit_v[pl.ds(d * N_EMBD + L * j, L)] for j in range(4)]

        @pl.when(jnp.logical_not(first))
        def _drain():
            pltpu.make_async_copy(ob, out_hbm.at[pl.ds(0, CB), 0], sem).wait()

        @plsc.parallel_loop(0, CB // L, unroll=2)
        def group_body(g):
            cvec = cnt_v[pl.ds(g * L, L)]
            for e in range(L):
                base = _lane_bcast(cvec, e) * N_EMBD
                vals = [plsc.load_gather(val_v, [base + (16 * j) + col0])
                        for j in range(4)]
                row = g * L + e
                for j in range(4):
                    ob[row, pl.ds(16 * j, L)] = vals[j] + bits[j]

        pltpu.async_copy(ob, out_hbm.at[pl.ds(b0, CB), d], sem)

    def pair_body(t2, carry):
        do_item(2 * t2, ob0, sem0, t2 == 0)
        do_item(2 * t2 + 1, ob1, sem1, t2 == 0)
        return carry

    lax.fori_loop(0, IPW // 2, pair_body, 0, unroll=False)

    pltpu.make_async_copy(ob0, out_hbm.at[pl.ds(0, CB), 0], sem0).wait()
    pltpu.make_async_copy(ob1, out_hbm.at[pl.ds(0, CB), 0], sem1).wait()


def kernel(count, val_emb, bit_emb):
    cnt_t = count.astype(jnp.int32).T
    val_flat = val_emb.reshape(-1)
    bit_flat = bit_emb.reshape(-1)

    mesh = plsc.VectorSubcoreMesh(core_axis_name="c", subcore_axis_name="s")
    f = pl.kernel(
        _sc_body,
        mesh=mesh,
        compiler_params=pltpu.CompilerParams(needs_layout_passes=False),
        out_type=jax.ShapeDtypeStruct((BATCH, COUNT_DIM, N_EMBD), jnp.float32),
        scratch_types=[
            pltpu.VMEM((NVALS * N_EMBD,), jnp.float32),
            pltpu.VMEM((COUNT_DIM * N_EMBD,), jnp.float32),
            pltpu.VMEM((CB,), jnp.int32),
            pltpu.VMEM((CB, N_EMBD), jnp.float32),
            pltpu.VMEM((CB, N_EMBD), jnp.float32),
            pltpu.SemaphoreType.DMA,
            pltpu.SemaphoreType.DMA,
        ],
    )
    return f(cnt_t, val_flat, bit_flat)
